# baseline (speedup 1.0000x reference)
import jax
import jax.numpy as jnp
from jax.experimental import pallas as pl
from jax.experimental.pallas import tpu as pltpu

_E = 1024
_BB = 16


def _vq_block(z_ref, emb_ref, emb2_ref, esq_ref, zq_ref, idx_ref, loss_ref):
    i = pl.program_id(0)
    nb = pl.num_programs(0)
    emb = emb_ref[...]
    emb2 = emb2_ref[...]
    esq = esq_ref[...]

    part = jnp.zeros((1, 1), jnp.float32)
    for b in range(_BB):
        z = z_ref[b]
        t = z.shape[1]
        zsq = jnp.sum(z * z, axis=0, keepdims=True)
        s2 = jax.lax.dot_general(emb2, z, (((1,), (0,)), ((), ())),
                                 preferred_element_type=jnp.float32)
        d = (zsq + esq) - s2
        m = jnp.min(d, axis=0, keepdims=True)
        io = jax.lax.broadcasted_iota(jnp.int32, (_E, t), 0)
        idx = jnp.min(jnp.where(d == m, io, _E), axis=0).astype(jnp.int32)

        oh = (io == idx[None, :]).astype(jnp.float32)
        zq = jax.lax.dot_general(emb, oh, (((0,), (0,)), ((), ())),
                                 preferred_element_type=jnp.float32)

        zq_ref[b] = z + (zq - z)
        idx_ref[0, b] = idx
        diff = zq - z
        part = part + jnp.sum(diff * diff).reshape(1, 1)

    @pl.when(i == 0)
    def _():
        loss_ref[...] = part

    @pl.when(i > 0)
    def _():
        loss_ref[...] = loss_ref[...] + part

    @pl.when(i == nb - 1)
    def _():
        loss_ref[...] = loss_ref[...] / (nb * _BB * 64 * 576)


def kernel(z_e, emb_weight):
    B, D, T = z_e.shape
    z32 = z_e.astype(jnp.float32)
    esq = jnp.sum(emb_weight ** 2, axis=1)[:, None]
    emb2 = emb_weight * 2.0

    zq, idx3, loss = pl.pallas_call(
        _vq_block,
        grid=(B // _BB,),
        in_specs=[
            pl.BlockSpec((_BB, D, T), lambda i: (i, 0, 0)),
            pl.BlockSpec((_E, D), lambda i: (0, 0)),
            pl.BlockSpec((_E, D), lambda i: (0, 0)),
            pl.BlockSpec((_E, 1), lambda i: (0, 0)),
        ],
        out_specs=[
            pl.BlockSpec((_BB, D, T), lambda i: (i, 0, 0)),
            pl.BlockSpec((1, _BB, T), lambda i: (i, 0, 0)),
            pl.BlockSpec((1, 1), lambda i: (0, 0)),
        ],
        out_shape=[
            jax.ShapeDtypeStruct((B, D, T), jnp.float32),
            jax.ShapeDtypeStruct((B // _BB, _BB, T), jnp.int32),
            jax.ShapeDtypeStruct((1, 1), jnp.float32),
        ],
    )(z32, emb_weight, emb2, esq)

    z_q_out = zq.astype(z_e.dtype)
    e_loss = loss[0, 0]
    encoding_indices = idx3.reshape(B, T)
    return (z_q_out, e_loss, encoding_indices)


# speedup vs baseline: 1.0205x; 1.0205x over previous
import jax
import jax.numpy as jnp
from jax.experimental import pallas as pl
from jax.experimental.pallas import tpu as pltpu

_E = 1024
_BB = 8


def _vq_block(z_ref, emb_ref, emb2_ref, esq_ref, zq_ref, idx_ref, loss_ref):
    i = pl.program_id(0)
    nb = pl.num_programs(0)
    emb = emb_ref[...]
    emb2 = emb2_ref[...]
    esq = esq_ref[...]

    part = jnp.zeros((1, 1), jnp.float32)
    for b in range(_BB):
        z = z_ref[b]
        t = z.shape[1]
        zsq = jnp.sum(z * z, axis=0, keepdims=True)
        s2 = jax.lax.dot_general(emb2, z, (((1,), (0,)), ((), ())),
                                 preferred_element_type=jnp.float32)
        d = (zsq + esq) - s2
        m = jnp.min(d, axis=0, keepdims=True)
        io = jax.lax.broadcasted_iota(jnp.int32, (_E, t), 0)
        idx = jnp.min(jnp.where(d == m, io, _E), axis=0).astype(jnp.int32)

        oh = (io == idx[None, :]).astype(jnp.float32)
        zq = jax.lax.dot_general(emb, oh, (((0,), (0,)), ((), ())),
                                 preferred_element_type=jnp.float32)

        zq_ref[b] = z + (zq - z)
        idx_ref[0, b] = idx
        diff = zq - z
        part = part + jnp.sum(diff * diff).reshape(1, 1)

    @pl.when(i == 0)
    def _():
        loss_ref[...] = part

    @pl.when(i > 0)
    def _():
        loss_ref[...] = loss_ref[...] + part

    @pl.when(i == nb - 1)
    def _():
        loss_ref[...] = loss_ref[...] / (nb * _BB * 64 * 576)


def kernel(z_e, emb_weight):
    B, D, T = z_e.shape
    z32 = z_e.astype(jnp.float32)
    esq = jnp.sum(emb_weight ** 2, axis=1)[:, None]
    emb2 = emb_weight * 2.0

    zq, idx3, loss = pl.pallas_call(
        _vq_block,
        grid=(B // _BB,),
        in_specs=[
            pl.BlockSpec((_BB, D, T), lambda i: (i, 0, 0)),
            pl.BlockSpec((_E, D), lambda i: (0, 0)),
            pl.BlockSpec((_E, D), lambda i: (0, 0)),
            pl.BlockSpec((_E, 1), lambda i: (0, 0)),
        ],
        out_specs=[
            pl.BlockSpec((_BB, D, T), lambda i: (i, 0, 0)),
            pl.BlockSpec((1, _BB, T), lambda i: (i, 0, 0)),
            pl.BlockSpec((1, 1), lambda i: (0, 0)),
        ],
        out_shape=[
            jax.ShapeDtypeStruct((B, D, T), jnp.float32),
            jax.ShapeDtypeStruct((B // _BB, _BB, T), jnp.int32),
            jax.ShapeDtypeStruct((1, 1), jnp.float32),
        ],
    )(z32, emb_weight, emb2, esq)

    z_q_out = zq.astype(z_e.dtype)
    e_loss = loss[0, 0]
    encoding_indices = idx3.reshape(B, T)
    return (z_q_out, e_loss, encoding_indices)


# speedup vs baseline: 1.0496x; 1.0285x over previous
import jax
import jax.numpy as jnp
from jax.experimental import pallas as pl
from jax.experimental.pallas import tpu as pltpu

_E = 1024
_SL = 128
_BB = 8


def _vq_block(z_ref, emb_ref, emb2_ref, esq_ref, zq_ref, idx_ref, loss_ref):
    i = pl.program_id(0)
    nb = pl.num_programs(0)
    emb = emb_ref[...]
    emb2 = emb2_ref[...]
    esq = esq_ref[...]
    t = z_ref.shape[2]
    io = jax.lax.broadcasted_iota(jnp.int32, (_SL, t), 0)

    part = jnp.zeros((1, 1), jnp.float32)
    for b in range(_BB):
        z = z_ref[b]
        zsq = jnp.sum(z * z, axis=0, keepdims=True)
        s2 = jax.lax.dot_general(emb2, z, (((1,), (0,)), ((), ())),
                                 preferred_element_type=jnp.float32)

        m = None
        idx = None
        for k in range(_E // _SL):
            sl = slice(k * _SL, (k + 1) * _SL)
            d = (zsq + esq[sl]) - s2[sl]
            mk = jnp.min(d, axis=0, keepdims=True)
            ik = jnp.min(jnp.where(d == mk, io, _SL), axis=0) + (k * _SL)
            if m is None:
                m, idx = mk, ik
            else:
                better = mk < m
                idx = jnp.where(better[0], ik, idx)
                m = jnp.where(better, mk, m)
        idx = idx.astype(jnp.int32)

        zq = jnp.zeros((z.shape[0], t), jnp.float32)
        for k in range(_E // _SL):
            sl = slice(k * _SL, (k + 1) * _SL)
            oh = (io == (idx - k * _SL)[None, :]).astype(jnp.float32)
            zq = zq + jax.lax.dot_general(emb[sl], oh, (((0,), (0,)), ((), ())),
                                          preferred_element_type=jnp.float32)

        zq_ref[b] = z + (zq - z)
        idx_ref[0, b] = idx
        diff = zq - z
        part = part + jnp.sum(diff * diff).reshape(1, 1)

    @pl.when(i == 0)
    def _():
        loss_ref[...] = part

    @pl.when(i > 0)
    def _():
        loss_ref[...] = loss_ref[...] + part

    @pl.when(i == nb - 1)
    def _():
        loss_ref[...] = loss_ref[...] / (nb * _BB * 64 * 576)


def kernel(z_e, emb_weight):
    B, D, T = z_e.shape
    z32 = z_e.astype(jnp.float32)
    esq = jnp.sum(emb_weight ** 2, axis=1)[:, None]
    emb2 = emb_weight * 2.0

    zq, idx3, loss = pl.pallas_call(
        _vq_block,
        grid=(B // _BB,),
        in_specs=[
            pl.BlockSpec((_BB, D, T), lambda i: (i, 0, 0)),
            pl.BlockSpec((_E, D), lambda i: (0, 0)),
            pl.BlockSpec((_E, D), lambda i: (0, 0)),
            pl.BlockSpec((_E, 1), lambda i: (0, 0)),
        ],
        out_specs=[
            pl.BlockSpec((_BB, D, T), lambda i: (i, 0, 0)),
            pl.BlockSpec((1, _BB, T), lambda i: (i, 0, 0)),
            pl.BlockSpec((1, 1), lambda i: (0, 0)),
        ],
        out_shape=[
            jax.ShapeDtypeStruct((B, D, T), jnp.float32),
            jax.ShapeDtypeStruct((B // _BB, _BB, T), jnp.int32),
            jax.ShapeDtypeStruct((1, 1), jnp.float32),
        ],
    )(z32, emb_weight, emb2, esq)

    z_q_out = zq.astype(z_e.dtype)
    e_loss = loss[0, 0]
    encoding_indices = idx3.reshape(B, T)
    return (z_q_out, e_loss, encoding_indices)
